# baseline (speedup 1.0000x reference)
import functools

import jax
import jax.numpy as jnp
from jax import lax
from jax.experimental import pallas as pl
from jax.experimental.pallas import tpu as pltpu
from jax.experimental.pallas import tpu_sc as plsc

D_MODEL = 1024
NUM_WORKERS = 32
CHUNK = 16
NB = 4


def _gather_body(table_hbm, idx_hbm, out_hbm, idx_v, *rest):
    bufs = rest[:NB]
    gsems = rest[NB:2 * NB]
    wsems = rest[2 * NB:3 * NB]
    n_idx = idx_hbm.shape[0]
    b_per_w = n_idx // NUM_WORKERS
    nchunks = b_per_w // CHUNK
    nq = nchunks // NB
    wid = lax.axis_index("s") * 2 + lax.axis_index("c")
    base = wid * b_per_w
    pltpu.sync_copy(idx_hbm.at[pl.ds(base, b_per_w)], idx_v)

    def g_copy(off, buf, sem):
        return pltpu.make_async_copy(
            table_hbm.at[idx_v.at[pl.ds(off, CHUNK)]], buf, sem)

    def w_copy(off, buf, sem):
        return pltpu.make_async_copy(
            buf, out_hbm.at[pl.ds(base + off, CHUNK)], sem)

    for j in range(NB):
        g_copy(j * CHUNK, bufs[j], gsems[j]).start()

    def body(i, carry):
        a = i * NB * CHUNK
        for j in range(NB):
            off = a + j * CHUNK
            g_copy(off, bufs[j], gsems[j]).wait()
            w_copy(off, bufs[j], wsems[j]).start()

        @pl.when(i < nq - 1)
        def _():
            for j in range(NB):
                off = a + j * CHUNK
                w_copy(off, bufs[j], wsems[j]).wait()
                g_copy(off + NB * CHUNK, bufs[j], gsems[j]).start()

        return carry

    lax.fori_loop(0, nq, body, 0)
    for j in range(NB):
        w_copy((nchunks - NB + j) * CHUNK, bufs[j], wsems[j]).wait()


def kernel(x, pe_table):
    batch, seq_len = x.shape
    n = batch * seq_len
    idx = x.reshape(n).astype(jnp.int32)
    mesh = plsc.VectorSubcoreMesh(core_axis_name="c", subcore_axis_name="s")
    gather = functools.partial(
        pl.kernel,
        mesh=mesh,
        out_type=jax.ShapeDtypeStruct((n, D_MODEL), jnp.float32),
        scratch_types=(
            [pltpu.VMEM((n // NUM_WORKERS,), jnp.int32)]
            + [pltpu.VMEM((CHUNK, D_MODEL), jnp.float32) for _ in range(NB)]
            + [pltpu.SemaphoreType.DMA for _ in range(2 * NB)]
        ),
    )(_gather_body)
    out = gather(pe_table, idx)
    return out.reshape(batch, seq_len, D_MODEL)


# speedup vs baseline: 1.0289x; 1.0289x over previous
import functools

import jax
import jax.numpy as jnp
from jax import lax
from jax.experimental import pallas as pl
from jax.experimental.pallas import tpu as pltpu
from jax.experimental.pallas import tpu_sc as plsc

D_MODEL = 1024
NUM_WORKERS = 32
CHUNK = 32


def _gather_body(idx_hbm, table_hbm, out_hbm, idx_v, buf0, buf1,
                 g0, g1, w0, w1):
    batch, seq_len = idx_hbm.shape
    w_per_b = NUM_WORKERS // batch
    b_per_w = seq_len // w_per_b
    nchunks = b_per_w // CHUNK
    npairs = nchunks // 2
    wid = lax.axis_index("s") * 2 + lax.axis_index("c")
    b = wid // w_per_b
    row0 = (wid % w_per_b) * b_per_w
    pltpu.sync_copy(idx_hbm.at[b, pl.ds(row0, b_per_w)], idx_v)

    def g_copy(off, buf, sem):
        return pltpu.make_async_copy(
            table_hbm.at[idx_v.at[pl.ds(off, CHUNK)]], buf, sem)

    def w_copy(off, buf, sem):
        return pltpu.make_async_copy(
            buf, out_hbm.at[b, pl.ds(row0 + off, CHUNK)], sem)

    g_copy(0, buf0, g0).start()

    def pair(i, carry):
        a = 2 * i * CHUNK

        @pl.when(i > 0)
        def _():
            w_copy(a - CHUNK, buf1, w1).wait()

        g_copy(a + CHUNK, buf1, g1).start()
        g_copy(a, buf0, g0).wait()
        w_copy(a, buf0, w0).start()

        @pl.when(i < npairs - 1)
        def _():
            w_copy(a, buf0, w0).wait()
            g_copy(a + 2 * CHUNK, buf0, g0).start()

        g_copy(a + CHUNK, buf1, g1).wait()
        w_copy(a + CHUNK, buf1, w1).start()
        return carry

    lax.fori_loop(0, npairs, pair, 0)
    last = (nchunks - 2) * CHUNK
    w_copy(last, buf0, w0).wait()
    w_copy(last + CHUNK, buf1, w1).wait()


def kernel(x, pe_table):
    batch, seq_len = x.shape
    mesh = plsc.VectorSubcoreMesh(core_axis_name="c", subcore_axis_name="s")
    gather = functools.partial(
        pl.kernel,
        mesh=mesh,
        out_type=jax.ShapeDtypeStruct((batch, seq_len, D_MODEL), jnp.float32),
        scratch_types=[
            pltpu.VMEM((seq_len * batch // NUM_WORKERS,), jnp.int32),
            pltpu.VMEM((CHUNK, D_MODEL), jnp.float32),
            pltpu.VMEM((CHUNK, D_MODEL), jnp.float32),
            pltpu.SemaphoreType.DMA,
            pltpu.SemaphoreType.DMA,
            pltpu.SemaphoreType.DMA,
            pltpu.SemaphoreType.DMA,
        ],
    )(_gather_body)
    return gather(x, pe_table)
